# baseline (speedup 1.0000x reference)
import jax
import jax.numpy as jnp
from jax import lax
from jax.experimental import pallas as pl
from jax.experimental.pallas import tpu as pltpu
from jax.experimental.pallas import tpu_sc as plsc

_N = 100000
_D = 128
_H = 10000
_GS = 32

_NC, _NS = 2, 16
_NW = _NC * _NS
_HPW = 320
_HPAD = _NW * _HPW
_CH = 2
_NCHUNK = _HPW // _CH
_ROWS = _CH * _GS
_NV = _D // 16
_K = 4


def _mm_body(ne_ref, wt_ref, b_ref, x_ref):
    x_ref[...] = jnp.maximum(
        jnp.dot(ne_ref[...], wt_ref[...], preferred_element_type=jnp.float32)
        + b_ref[...], 0.0)


def _transform(ne, wt, b):
    bn = 1000
    return pl.pallas_call(
        _mm_body,
        grid=(_N // bn,),
        in_specs=[
            pl.BlockSpec((bn, _D), lambda i: (i, 0)),
            pl.BlockSpec((_D, _D), lambda i: (0, 0)),
            pl.BlockSpec((1, _D), lambda i: (0, 0)),
        ],
        out_specs=pl.BlockSpec((bn, _D), lambda i: (i, 0)),
        out_shape=jax.ShapeDtypeStruct((_N, _D), jnp.float32),
    )(ne, wt, b.reshape(1, _D))


def _sc_body(x_hbm, idx_hbm, out_hbm, idx_v, buf, out_v,
             sem0, sem1, sem2, sem3):
    wid = lax.axis_index("s") * _NC + lax.axis_index("c")
    sems = (sem0, sem1, sem2, sem3)
    pltpu.sync_copy(idx_hbm.at[wid], idx_v)

    for b in range(_K):
        pltpu.async_copy(x_hbm.at[idx_v.at[b]], buf.at[b], sems[b])

    def reduce_chunk(c, b):
        for h in range(_CH):
            base = h * _GS
            row = c * _CH + h
            for d in range(_NV):
                off = d * 16
                vals = [buf[b, base + r, pl.ds(off, 16)] for r in range(_GS)]
                while len(vals) > 1:
                    vals = [vals[i] + vals[i + 1]
                            for i in range(0, len(vals), 2)]
                out_v[row, pl.ds(off, 16)] = vals[0] * (1.0 / _GS)

    def group(g, carry):
        for b in range(_K):
            c = g * _K + b
            pltpu.make_async_copy(
                x_hbm.at[idx_v.at[c]], buf.at[b], sems[b]).wait()
        for b in range(_K):
            c = g * _K + b
            reduce_chunk(c, b)
            pltpu.async_copy(
                x_hbm.at[idx_v.at[c + _K]], buf.at[b], sems[b])
        return carry

    lax.fori_loop(0, _NCHUNK // _K, group, 0)
    for b in range(_K):
        pltpu.make_async_copy(
            x_hbm.at[idx_v.at[_NCHUNK + b]], buf.at[b], sems[b]).wait()
    pltpu.sync_copy(out_v, out_hbm.at[pl.ds(wid * _HPW, _HPW)])


def _aggregate(x, idx):
    mesh = plsc.VectorSubcoreMesh(core_axis_name="c", subcore_axis_name="s")
    k = pl.kernel(
        _sc_body,
        out_type=jax.ShapeDtypeStruct((_HPAD, _D), jnp.float32),
        mesh=mesh,
        scratch_types=[
            pltpu.VMEM((_NCHUNK + _K, _ROWS), jnp.int32),
            pltpu.VMEM((_K, _ROWS, _D), jnp.float32),
            pltpu.VMEM((_HPW, _D), jnp.float32),
            pltpu.SemaphoreType.DMA,
            pltpu.SemaphoreType.DMA,
            pltpu.SemaphoreType.DMA,
            pltpu.SemaphoreType.DMA,
        ],
    )
    return k(x, idx)


def kernel(node_embeddings, hyperedges, hyperedge_subsets, W, b):
    del hyperedges
    x = _transform(node_embeddings, W.T, b)
    idx = hyperedge_subsets.astype(jnp.int32).reshape(_H, _GS)
    idx = jnp.pad(idx, ((0, _HPAD - _H), (0, 0)))
    idx = idx.reshape(_NW, _NCHUNK, _ROWS)
    idx = jnp.pad(idx, ((0, 0), (0, _K), (0, 0)))
    return _aggregate(x, idx)[:_H]


# speedup vs baseline: 1.3702x; 1.3702x over previous
import jax
import jax.numpy as jnp
from jax import lax
from jax.experimental import pallas as pl
from jax.experimental.pallas import tpu as pltpu
from jax.experimental.pallas import tpu_sc as plsc

_N = 100000
_D = 128
_H = 10000
_GS = 32

_NC, _NS = 2, 16
_NW = _NC * _NS
_HPW = 320
_HPAD = _NW * _HPW
_CH = 8
_NCHUNK = _HPW // _CH
_IR = _CH * _GS // 128
_NV = _D // 16


def _mm_body(ne_ref, wt_ref, b_ref, x_ref):
    x_ref[...] = jnp.maximum(
        jnp.dot(ne_ref[...], wt_ref[...], preferred_element_type=jnp.float32)
        + b_ref[...], 0.0)


def _transform(ne, wt, b):
    bn = 1000
    return pl.pallas_call(
        _mm_body,
        grid=(_N // bn,),
        in_specs=[
            pl.BlockSpec((bn, _D), lambda i: (i, 0)),
            pl.BlockSpec((_D, _D), lambda i: (0, 0)),
            pl.BlockSpec((1, _D), lambda i: (0, 0)),
        ],
        out_specs=pl.BlockSpec((bn, _D), lambda i: (i, 0)),
        out_shape=jax.ShapeDtypeStruct((_N, _D), jnp.float32),
    )(ne, wt, b.reshape(1, _D))


def _sc_body(x_hbm, idx_hbm, out_hbm, idx_v, buf, out_v, sem):
    wid = lax.axis_index("s") * _NC + lax.axis_index("c")
    pltpu.sync_copy(idx_hbm.at[wid], idx_v)

    def reduce_chunk(c):
        for h in range(_CH):
            base = h * _GS
            row = c * _CH + h
            for d in range(_NV):
                off = d * 16
                vals = [buf[base + r, pl.ds(off, 16)]
                        for r in range(_GS)]
                while len(vals) > 1:
                    vals = [vals[i] + vals[i + 1]
                            for i in range(0, len(vals), 2)]
                out_v[row, pl.ds(off, 16)] = vals[0] * (1.0 / _GS)

    def chunk(c, carry):
        pltpu.async_copy(
            x_hbm.at[idx_v.at[pl.ds(c * _CH * _GS, _CH * _GS)]],
            buf, sem).wait()
        reduce_chunk(c)
        return carry

    lax.fori_loop(0, _NCHUNK, chunk, 0)
    pltpu.sync_copy(out_v, out_hbm.at[pl.ds(wid * _HPW, _HPW)])


def _aggregate(x, idx):
    mesh = plsc.VectorSubcoreMesh(core_axis_name="c", subcore_axis_name="s")
    k = pl.kernel(
        _sc_body,
        out_type=jax.ShapeDtypeStruct((_HPAD, _D), jnp.float32),
        mesh=mesh,
        scratch_types=[
            pltpu.VMEM((_NCHUNK * _CH * _GS,), jnp.int32),
            pltpu.VMEM((_CH * _GS, _D), jnp.float32),
            pltpu.VMEM((_HPW, _D), jnp.float32),
            pltpu.SemaphoreType.DMA,
        ],
    )
    return k(x, idx)


def kernel(node_embeddings, hyperedges, hyperedge_subsets, W, b):
    del hyperedges
    x = _transform(node_embeddings, W.T, b)
    idx = hyperedge_subsets.astype(jnp.int32).reshape(_H, _GS)
    idx = jnp.pad(idx, ((0, _HPAD - _H), (0, 0)))
    idx = idx.reshape(_NW, _NCHUNK * _CH * _GS)
    return _aggregate(x, idx)[:_H]


# speedup vs baseline: 1.6268x; 1.1873x over previous
import jax
import jax.numpy as jnp
from jax import lax
from jax.experimental import pallas as pl
from jax.experimental.pallas import tpu as pltpu
from jax.experimental.pallas import tpu_sc as plsc

_N = 100000
_D = 128
_H = 10000
_GS = 32

_NC, _NS = 2, 16
_CH = 8
_CR = _CH * _GS
_NV = _D // 16

_NCH0 = 57
_NCH1 = 23
_HPW0 = _NCH0 * _CH
_HPW1 = _NCH1 * _CH
_HPS = _HPW0 + _HPW1
_HPAD = _NS * _HPS
_HIDX = _HPAD + _HPW0 - _HPW1


def _mm_body(ne_ref, wt_ref, b_ref, x_ref):
    x_ref[...] = jnp.maximum(
        jnp.dot(ne_ref[...], wt_ref[...], preferred_element_type=jnp.float32)
        + b_ref[...], 0.0)


def _transform(ne, wt, b):
    bn = 1000
    return pl.pallas_call(
        _mm_body,
        grid=(_N // bn,),
        in_specs=[
            pl.BlockSpec((bn, _D), lambda i: (i, 0)),
            pl.BlockSpec((_D, _D), lambda i: (0, 0)),
            pl.BlockSpec((1, _D), lambda i: (0, 0)),
        ],
        out_specs=pl.BlockSpec((bn, _D), lambda i: (i, 0)),
        out_shape=jax.ShapeDtypeStruct((_N, _D), jnp.float32),
    )(ne, wt, b.reshape(1, _D))


def _sc_body(x_hbm, idx_hbm, out_hbm, idx_v, buf, out_v, sem):
    c = lax.axis_index("c")
    s = lax.axis_index("s")
    off = s * _HPS + c * _HPW0
    nch = _NCH0 - c * (_NCH0 - _NCH1)
    pltpu.sync_copy(idx_hbm.at[pl.ds(off * _GS, _NCH0 * _CR)], idx_v)

    def reduce_chunk(k):
        for h in range(_CH):
            base = h * _GS
            row = k * _CH + h
            for d in range(_NV):
                o = d * 16
                vals = [buf[base + r, pl.ds(o, 16)] for r in range(_GS)]
                while len(vals) > 1:
                    vals = [vals[i] + vals[i + 1]
                            for i in range(0, len(vals), 2)]
                out_v[row, pl.ds(o, 16)] = vals[0] * (1.0 / _GS)

    def chunk(k, carry):
        pltpu.async_copy(
            x_hbm.at[idx_v.at[pl.ds(k * _CR, _CR)]], buf, sem).wait()
        reduce_chunk(k)
        return carry

    lax.fori_loop(0, nch, chunk, 0)

    @pl.when(c == 0)
    def _():
        pltpu.sync_copy(out_v, out_hbm.at[pl.ds(off, _HPW0)])

    @pl.when(c == 1)
    def _():
        pltpu.sync_copy(out_v.at[pl.ds(0, _HPW1)],
                        out_hbm.at[pl.ds(off, _HPW1)])


def _aggregate(x, idx):
    mesh = plsc.VectorSubcoreMesh(core_axis_name="c", subcore_axis_name="s")
    k = pl.kernel(
        _sc_body,
        out_type=jax.ShapeDtypeStruct((_HPAD, _D), jnp.float32),
        mesh=mesh,
        scratch_types=[
            pltpu.VMEM((_NCH0 * _CR,), jnp.int32),
            pltpu.VMEM((_CR, _D), jnp.float32),
            pltpu.VMEM((_HPW0, _D), jnp.float32),
            pltpu.SemaphoreType.DMA,
        ],
    )
    return k(x, idx)


def kernel(node_embeddings, hyperedges, hyperedge_subsets, W, b):
    del hyperedges
    x = _transform(node_embeddings, W.T, b)
    idx = hyperedge_subsets.astype(jnp.int32).reshape(_H, _GS)
    idx = jnp.pad(idx, ((0, _HIDX - _H), (0, 0)))
    idx = idx.reshape(_HIDX * _GS)
    return _aggregate(x, idx)[:_H]


# speedup vs baseline: 1.6409x; 1.0087x over previous
import jax
import jax.numpy as jnp
from jax import lax
from jax.experimental import pallas as pl
from jax.experimental.pallas import tpu as pltpu
from jax.experimental.pallas import tpu_sc as plsc

_N = 100000
_D = 128
_H = 10000
_GS = 32

_NC, _NS = 2, 16
_CH = 8
_CR = _CH * _GS
_NV = _D // 16

_NCH0 = 56
_NCH1 = 24
_HPW0 = _NCH0 * _CH
_HPW1 = _NCH1 * _CH
_HPS = _HPW0 + _HPW1
_HPAD = _NS * _HPS
_HIDX = _HPAD + _HPW0 - _HPW1


def _mm_body(ne_ref, wt_ref, b_ref, x_ref):
    x_ref[...] = jnp.maximum(
        jnp.dot(ne_ref[...], wt_ref[...], preferred_element_type=jnp.float32)
        + b_ref[...], 0.0)


def _transform(ne, wt, b):
    bn = 1000
    return pl.pallas_call(
        _mm_body,
        grid=(_N // bn,),
        in_specs=[
            pl.BlockSpec((bn, _D), lambda i: (i, 0)),
            pl.BlockSpec((_D, _D), lambda i: (0, 0)),
            pl.BlockSpec((1, _D), lambda i: (0, 0)),
        ],
        out_specs=pl.BlockSpec((bn, _D), lambda i: (i, 0)),
        out_shape=jax.ShapeDtypeStruct((_N, _D), jnp.float32),
    )(ne, wt, b.reshape(1, _D))


def _sc_body(x_hbm, idx_hbm, out_hbm, idx_v, buf, out_v, sem, sem2):
    c = lax.axis_index("c")
    s = lax.axis_index("s")
    off = s * _HPS + c * _HPW0
    nch = _NCH0 - c * (_NCH0 - _NCH1)
    pltpu.sync_copy(idx_hbm.at[pl.ds(off * _GS, _NCH0 * _CR)], idx_v)

    def reduce_chunk(k):
        for h in range(_CH):
            base = h * _GS
            row = k * _CH + h
            for d in range(_NV):
                o = d * 16
                vals = [buf[base + r, pl.ds(o, 16)] for r in range(_GS)]
                while len(vals) > 1:
                    vals = [vals[i] + vals[i + 1]
                            for i in range(0, len(vals), 2)]
                out_v[row, pl.ds(o, 16)] = vals[0] * (1.0 / _GS)

    half = _CR // 2

    def chunk(k, carry):
        cp1 = pltpu.async_copy(
            x_hbm.at[idx_v.at[pl.ds(k * _CR, half)]],
            buf.at[pl.ds(0, half)], sem)
        cp2 = pltpu.async_copy(
            x_hbm.at[idx_v.at[pl.ds(k * _CR + half, half)]],
            buf.at[pl.ds(half, half)], sem2)
        cp1.wait()
        cp2.wait()
        reduce_chunk(k)
        return carry

    lax.fori_loop(0, nch, chunk, 0)

    @pl.when(c == 0)
    def _():
        pltpu.sync_copy(out_v, out_hbm.at[pl.ds(off, _HPW0)])

    @pl.when(c == 1)
    def _():
        pltpu.sync_copy(out_v.at[pl.ds(0, _HPW1)],
                        out_hbm.at[pl.ds(off, _HPW1)])


def _aggregate(x, idx):
    mesh = plsc.VectorSubcoreMesh(core_axis_name="c", subcore_axis_name="s")
    k = pl.kernel(
        _sc_body,
        out_type=jax.ShapeDtypeStruct((_HPAD, _D), jnp.float32),
        mesh=mesh,
        scratch_types=[
            pltpu.VMEM((_NCH0 * _CR,), jnp.int32),
            pltpu.VMEM((_CR, _D), jnp.float32),
            pltpu.VMEM((_HPW0, _D), jnp.float32),
            pltpu.SemaphoreType.DMA,
            pltpu.SemaphoreType.DMA,
        ],
    )
    return k(x, idx)


def kernel(node_embeddings, hyperedges, hyperedge_subsets, W, b):
    del hyperedges
    x = _transform(node_embeddings, W.T, b)
    idx = hyperedge_subsets.astype(jnp.int32).reshape(_H, _GS)
    idx = jnp.pad(idx, ((0, _HIDX - _H), (0, 0)))
    idx = idx.reshape(_HIDX * _GS)
    return _aggregate(x, idx)[:_H]


# speedup vs baseline: 1.7132x; 1.0441x over previous
import jax
import jax.numpy as jnp
from jax import lax
from jax.experimental import pallas as pl
from jax.experimental.pallas import tpu as pltpu
from jax.experimental.pallas import tpu_sc as plsc

_N = 100000
_D = 128
_H = 10000
_GS = 32

_NC, _NS = 2, 16
_CH = 8
_CR = _CH * _GS
_NV = _D // 16

_NCH0 = 56
_NCH1 = 24
_HPW0 = _NCH0 * _CH
_HPW1 = _NCH1 * _CH
_HPS = _HPW0 + _HPW1
_HPAD = _NS * _HPS
_HIDX = _HPAD + _HPW0 - _HPW1


def _mm_body(ne_ref, wt_ref, b_ref, x_ref):
    x_ref[...] = jnp.maximum(
        jnp.dot(ne_ref[...].astype(jnp.bfloat16),
                wt_ref[...].astype(jnp.bfloat16),
                preferred_element_type=jnp.float32)
        + b_ref[...], 0.0)


def _transform(ne, wt, b):
    bn = 2000
    return pl.pallas_call(
        _mm_body,
        grid=(_N // bn,),
        in_specs=[
            pl.BlockSpec((bn, _D), lambda i: (i, 0)),
            pl.BlockSpec((_D, _D), lambda i: (0, 0)),
            pl.BlockSpec((1, _D), lambda i: (0, 0)),
        ],
        out_specs=pl.BlockSpec((bn, _D), lambda i: (i, 0)),
        out_shape=jax.ShapeDtypeStruct((_N, _D), jnp.float32),
    )(ne, wt, b.reshape(1, _D))


def _sc_body(x_hbm, idx_hbm, out_hbm, idx_v, buf, out_v, sem, sem2):
    c = lax.axis_index("c")
    s = lax.axis_index("s")
    off = s * _HPS + c * _HPW0
    nch = _NCH0 - c * (_NCH0 - _NCH1)
    pltpu.sync_copy(idx_hbm.at[pl.ds(off * _GS, _NCH0 * _CR)], idx_v)

    def reduce_chunk(k):
        for h in range(_CH):
            base = h * _GS
            row = k * _CH + h
            for d in range(_NV):
                o = d * 16
                vals = [buf[base + r, pl.ds(o, 16)] for r in range(_GS)]
                while len(vals) > 1:
                    vals = [vals[i] + vals[i + 1]
                            for i in range(0, len(vals), 2)]
                out_v[row, pl.ds(o, 16)] = vals[0] * (1.0 / _GS)

    half = _CR // 2

    def chunk(k, carry):
        cp1 = pltpu.async_copy(
            x_hbm.at[idx_v.at[pl.ds(k * _CR, half)]],
            buf.at[pl.ds(0, half)], sem)
        cp2 = pltpu.async_copy(
            x_hbm.at[idx_v.at[pl.ds(k * _CR + half, half)]],
            buf.at[pl.ds(half, half)], sem2)
        cp1.wait()
        cp2.wait()
        reduce_chunk(k)
        return carry

    lax.fori_loop(0, nch, chunk, 0)

    @pl.when(c == 0)
    def _():
        pltpu.sync_copy(out_v, out_hbm.at[pl.ds(off, _HPW0)])

    @pl.when(c == 1)
    def _():
        pltpu.sync_copy(out_v.at[pl.ds(0, _HPW1)],
                        out_hbm.at[pl.ds(off, _HPW1)])


def _aggregate(x, idx):
    mesh = plsc.VectorSubcoreMesh(core_axis_name="c", subcore_axis_name="s")
    k = pl.kernel(
        _sc_body,
        out_type=jax.ShapeDtypeStruct((_HPAD, _D), jnp.float32),
        mesh=mesh,
        scratch_types=[
            pltpu.VMEM((_NCH0 * _CR,), jnp.int32),
            pltpu.VMEM((_CR, _D), jnp.float32),
            pltpu.VMEM((_HPW0, _D), jnp.float32),
            pltpu.SemaphoreType.DMA,
            pltpu.SemaphoreType.DMA,
        ],
    )
    return k(x, idx)


def kernel(node_embeddings, hyperedges, hyperedge_subsets, W, b):
    del hyperedges
    x = _transform(node_embeddings, W.T, b)
    idx = hyperedge_subsets.astype(jnp.int32).reshape(_H, _GS)
    idx = jnp.pad(idx, ((0, _HIDX - _H), (0, 0)))
    idx = idx.reshape(_HIDX * _GS)
    return _aggregate(x, idx)[:_H]
